# Initial kernel scaffold; baseline (speedup 1.0000x reference)
import jax
import jax.numpy as jnp
from jax.experimental import pallas as pl


def kernel(id_feature, dense_feature, base_embedding, W1, b1, W2, b2, W3, b3):
    raise NotImplementedError("write your pallas kernel here")


# speedup vs baseline: 6.2445x; 6.2445x over previous
import functools

import jax
import jax.numpy as jnp
from jax import lax
from jax.experimental import pallas as pl
from jax.experimental.pallas import tpu as pltpu
from jax.experimental.pallas import tpu_sc as plsc

ID_LEN = 26
DENSE_LEN = 26
N_ID = 13
N_DENSE = 13
EMB = 32
BATCH = 16384
VOCAB = 2000

B13 = BATCH * N_ID


def _make_sc_gather(n_rows: int, emb: int):
    info = plsc.get_sparse_core_info()
    nw = info.num_cores * info.num_subcores
    assert n_rows % nw == 0
    rows_per_w = n_rows // nw
    chunk = rows_per_w
    n_chunks = 1
    while chunk * emb * 4 > 128 * 1024:
        n_chunks *= 2
        chunk = rows_per_w // n_chunks
    assert chunk * n_chunks == rows_per_w and chunk % 8 == 0

    mesh = plsc.VectorSubcoreMesh(core_axis_name="c", subcore_axis_name="s")

    @functools.partial(
        pl.kernel,
        mesh=mesh,
        out_type=jax.ShapeDtypeStruct((n_rows, emb), jnp.float32),
        scratch_types=[
            pltpu.VMEM((chunk,), jnp.int32),
            pltpu.VMEM((chunk, emb), jnp.float32),
            pltpu.SemaphoreType.DMA,
        ],
        compiler_params=pltpu.CompilerParams(use_tc_tiling_on_sc=False),
    )
    def gather_k(table_hbm, idx_hbm, out_hbm, idx_v, rows_v, sem):
        wid = lax.axis_index("s") * info.num_cores + lax.axis_index("c")
        base = wid * rows_per_w
        for c in range(n_chunks):
            off = base + c * chunk
            pltpu.sync_copy(idx_hbm.at[pl.ds(off, chunk)], idx_v)
            pltpu.async_copy(table_hbm.at[idx_v], rows_v, sem).wait()
            pltpu.sync_copy(rows_v, out_hbm.at[pl.ds(off, chunk)])

    return gather_k


@functools.lru_cache(maxsize=None)
def _sc_gather_cached():
    return _make_sc_gather(B13, EMB)


def _mlp_body(e_ref, d_ref, w1a_ref, w1d_ref, b1_ref, w2_ref, b2_ref,
              w3_ref, b3_ref, out_ref):
    x = (jnp.dot(e_ref[...], w1a_ref[...], preferred_element_type=jnp.float32)
         + jnp.dot(d_ref[...], w1d_ref[...], preferred_element_type=jnp.float32)
         + b1_ref[...])
    h = jnp.maximum(x, 0.0)
    h = jnp.maximum(
        jnp.dot(h, w2_ref[...], preferred_element_type=jnp.float32)
        + b2_ref[...], 0.0)
    out_ref[...] = (
        jnp.dot(h, w3_ref[...], preferred_element_type=jnp.float32)
        + b3_ref[...])


def _mlp(emb_mat, d, w1a, w1d, b1, w2, b2, w3, b3, tb: int = 1024):
    batch = emb_mat.shape[0]
    grid = (batch // tb,)
    full = lambda shape: pl.BlockSpec(shape, lambda i: (0, 0))
    return pl.pallas_call(
        _mlp_body,
        grid=grid,
        in_specs=[
            pl.BlockSpec((tb, emb_mat.shape[1]), lambda i: (i, 0)),
            pl.BlockSpec((tb, d.shape[1]), lambda i: (i, 0)),
            full(w1a.shape),
            full(w1d.shape),
            full(b1.shape),
            full(w2.shape),
            full(b2.shape),
            full(w3.shape),
            full(b3.shape),
        ],
        out_specs=pl.BlockSpec((tb, 1), lambda i: (i, 0)),
        out_shape=jax.ShapeDtypeStruct((batch, 1), jnp.float32),
    )(emb_mat, d, w1a, w1d, b1, w2, b2, w3, b3)


def kernel(id_feature, dense_feature, base_embedding, W1, b1, W2, b2, W3, b3):
    ids = id_feature[:, :N_ID].reshape(-1).astype(jnp.int32)
    d = dense_feature[:, -N_DENSE:]
    ew = N_ID * EMB
    w1a = W1[:ew] + W1[ew:2 * ew]
    w1d = W1[2 * ew:2 * ew + N_DENSE] + W1[2 * ew + N_DENSE:]

    rows = _sc_gather_cached()(base_embedding, ids)
    emb_mat = rows.reshape(BATCH, N_ID * EMB)

    return _mlp(emb_mat, d, w1a, w1d,
                b1.reshape(1, -1), W2, b2.reshape(1, -1),
                W3, b3.reshape(1, -1))
